# Initial kernel scaffold; baseline (speedup 1.0000x reference)
import jax
import jax.numpy as jnp
from jax.experimental import pallas as pl


def kernel(x, W, b):
    raise NotImplementedError("write your pallas kernel here")


# speedup vs baseline: 22.8228x; 22.8228x over previous
import jax
import jax.numpy as jnp
from jax.experimental import pallas as pl

B = 4096
IN_DIM = 128
OUT_DIM = 256
N_BINS = 50
TAU = 10.0

BS = 256


def _encoder_block(x_ref, w_ref, b_ref, lat_ref, spk_ref):
    rates = jax.lax.dot_general(
        x_ref[...], w_ref[...],
        dimension_numbers=(((1,), (1,)), ((), ())),
        preferred_element_type=jnp.float32,
    ) + b_ref[...]
    lat = jnp.clip(N_BINS * jnp.exp(-rates / TAU), 1, N_BINS - 1).astype(jnp.int32)
    lat_ref[...] = lat
    bins = jax.lax.broadcasted_iota(jnp.int32, (BS, N_BINS, OUT_DIM), 1)
    spk_ref[...] = (bins == lat[:, None, :]).astype(jnp.float32)


def kernel(x, W, b):
    b2 = b.reshape(1, OUT_DIM)
    grid = (B // BS,)
    lat, spikes = pl.pallas_call(
        _encoder_block,
        grid=grid,
        in_specs=[
            pl.BlockSpec((BS, IN_DIM), lambda i: (i, 0)),
            pl.BlockSpec((OUT_DIM, IN_DIM), lambda i: (0, 0)),
            pl.BlockSpec((1, OUT_DIM), lambda i: (0, 0)),
        ],
        out_specs=[
            pl.BlockSpec((BS, OUT_DIM), lambda i: (i, 0)),
            pl.BlockSpec((BS, N_BINS, OUT_DIM), lambda i: (i, 0, 0)),
        ],
        out_shape=[
            jax.ShapeDtypeStruct((B, OUT_DIM), jnp.int32),
            jax.ShapeDtypeStruct((B, N_BINS, OUT_DIM), jnp.float32),
        ],
    )(x, W, b2)
    return (lat, spikes)


# speedup vs baseline: 22.8942x; 1.0031x over previous
import jax
import jax.numpy as jnp
from jax.experimental import pallas as pl

B = 4096
IN_DIM = 128
OUT_DIM = 256
N_BINS = 50
TAU = 10.0

BS = 128


def _encoder_block(x_ref, w_ref, b_ref, lat_ref, spk_ref):
    rates = jax.lax.dot_general(
        x_ref[...], w_ref[...],
        dimension_numbers=(((1,), (1,)), ((), ())),
        preferred_element_type=jnp.float32,
    ) + b_ref[...]
    lat = jnp.clip(N_BINS * jnp.exp(-rates / TAU), 1, N_BINS - 1).astype(jnp.int32)
    lat_ref[...] = lat
    bins = jax.lax.broadcasted_iota(jnp.int32, (BS, N_BINS, OUT_DIM), 1)
    spk_ref[...] = (bins == lat[:, None, :]).astype(jnp.float32)


def kernel(x, W, b):
    b2 = b.reshape(1, OUT_DIM)
    grid = (B // BS,)
    lat, spikes = pl.pallas_call(
        _encoder_block,
        grid=grid,
        in_specs=[
            pl.BlockSpec((BS, IN_DIM), lambda i: (i, 0)),
            pl.BlockSpec((OUT_DIM, IN_DIM), lambda i: (0, 0)),
            pl.BlockSpec((1, OUT_DIM), lambda i: (0, 0)),
        ],
        out_specs=[
            pl.BlockSpec((BS, OUT_DIM), lambda i: (i, 0)),
            pl.BlockSpec((BS, N_BINS, OUT_DIM), lambda i: (i, 0, 0)),
        ],
        out_shape=[
            jax.ShapeDtypeStruct((B, OUT_DIM), jnp.int32),
            jax.ShapeDtypeStruct((B, N_BINS, OUT_DIM), jnp.float32),
        ],
    )(x, W, b2)
    return (lat, spikes)
